# baseline (speedup 1.0000x reference)
import functools
import jax
import jax.numpy as jnp
from jax import lax
from jax.experimental import pallas as pl
from jax.experimental.pallas import tpu as pltpu
from jax.experimental.pallas import tpu_sc as plsc

_B, _S, _D = 4, 8192, 1024
_NW = 32
_ROWS = _B * _S
_SEQ_W = _S // _NW
_T = 16
_NSC = _SEQ_W // _T


@functools.partial(
    pl.kernel,
    mesh=plsc.VectorSubcoreMesh(core_axis_name="c", subcore_axis_name="s"),
    out_type=jax.ShapeDtypeStruct((_ROWS, _D), jnp.float32),
    scratch_types=[
        pltpu.VMEM((2, _T, _D), jnp.float32),
        pltpu.VMEM((2, _T, _D), jnp.float32),
        pltpu.VMEM((2, _T, _D), jnp.float32),
        pltpu.SemaphoreType.DMA,
        pltpu.SemaphoreType.DMA,
        pltpu.SemaphoreType.DMA,
        pltpu.SemaphoreType.DMA,
        pltpu.SemaphoreType.DMA,
        pltpu.SemaphoreType.DMA,
    ],
)
def _sc_add(x_hbm, emb_hbm, out_hbm, xbuf, ebuf, obuf,
            sx0, sx1, se0, se1, so0, so1):
    sx, se, so = [sx0, sx1], [se0, se1], [so0, so1]
    wid = lax.axis_index("s") * 2 + lax.axis_index("c")
    seqbase = wid * _SEQ_W

    def x_slice(sc, b):
        return x_hbm.at[pl.ds(b * _S + seqbase + sc * _T, _T), :]

    o_slice = x_slice

    def e_slice(sc):
        return emb_hbm.at[pl.ds(seqbase + sc * _T, _T), :]

    pltpu.async_copy(e_slice(0), ebuf.at[0], se[0])
    pltpu.async_copy(x_slice(0, 0), xbuf.at[0], sx[0])
    pltpu.async_copy(x_slice(0, 1), xbuf.at[1], sx[1])

    def outer(g, _):
        for sp in range(2):
            sc = 2 * g + sp
            pltpu.make_async_copy(e_slice(sc), ebuf.at[sp], se[sp]).wait()

            @pl.when(sc < _NSC - 1)
            def _next_emb():
                pltpu.async_copy(e_slice(sc + 1), ebuf.at[1 - sp], se[1 - sp])

            for b in range(_B):
                bp = b % 2
                pltpu.make_async_copy(x_slice(sc, b), xbuf.at[bp],
                                      sx[bp]).wait()

                if b >= 2:
                    pltpu.make_async_copy(obuf.at[bp], o_slice(sc, b - 2),
                                          so[bp]).wait()
                else:
                    @pl.when(sc > 0)
                    def _wait_store():
                        prev_sc = sc - 1
                        pltpu.make_async_copy(obuf.at[bp],
                                              o_slice(prev_sc, b + 2),
                                              so[bp]).wait()

                def add_row(r, _):
                    for u in range(_D // 16):
                        sl = pl.ds(u * 16, 16)
                        obuf[bp, r, sl] = xbuf[bp, r, sl] + ebuf[sp, r, sl]
                    return 0

                lax.fori_loop(0, _T, add_row, 0)

                pltpu.async_copy(obuf.at[bp], o_slice(sc, b), so[bp])

                if b < 2:
                    pltpu.async_copy(x_slice(sc, b + 2), xbuf.at[bp], sx[bp])
                else:
                    @pl.when(sc < _NSC - 1)
                    def _next_x():
                        pltpu.async_copy(x_slice(sc + 1, b - 2), xbuf.at[bp],
                                         sx[bp])

        return 0

    lax.fori_loop(0, _NSC // 2, outer, 0)

    for b in (2, 3):
        pltpu.make_async_copy(obuf.at[b % 2], o_slice(_NSC - 1, b),
                              so[b % 2]).wait()


def kernel(x, emb):
    B, S, D = x.shape
    out = _sc_add(x.reshape(B * S, D), emb)
    return out.reshape(B, S, D)


# speedup vs baseline: 2.3173x; 2.3173x over previous
import jax
import jax.numpy as jnp
from jax.experimental import pallas as pl

_SEQ_BLK = 2048


def _add_kernel(x_ref, emb_ref, o_ref):
    o_ref[...] = x_ref[...] + emb_ref[...]


def kernel(x, emb):
    B, S, D = x.shape
    grid = (S // _SEQ_BLK, B)
    return pl.pallas_call(
        _add_kernel,
        grid=grid,
        in_specs=[
            pl.BlockSpec((1, _SEQ_BLK, D), lambda i, j: (j, i, 0)),
            pl.BlockSpec((_SEQ_BLK, D), lambda i, j: (i, 0)),
        ],
        out_specs=pl.BlockSpec((1, _SEQ_BLK, D), lambda i, j: (j, i, 0)),
        out_shape=jax.ShapeDtypeStruct((B, S, D), x.dtype),
    )(x, emb)
